# Initial kernel scaffold; baseline (speedup 1.0000x reference)
import jax
import jax.numpy as jnp
from jax.experimental import pallas as pl


def kernel(mu, anno, annotators, confidence, random_effects):
    raise NotImplementedError("write your pallas kernel here")


# speedup vs baseline: 9.3751x; 9.3751x over previous
import functools
import math

import jax
import jax.numpy as jnp
from jax import lax
from jax.experimental import pallas as pl
from jax.experimental.pallas import tpu as pltpu
from jax.experimental.pallas import tpu_sc as plsc

K = 8
D = 4
N = 262144
A = 16384
MIN_LL = math.log(1e-06)


_TBLK = 2048


def _table_body(mu_ref, re_ref, out_ref):
    emu = jnp.exp(mu_ref[0:1, :])
    z = emu + re_ref[...]
    x0 = z[:, 0:8]
    x1 = z[:, 8:16]
    x2 = z[:, 16:24]
    x3 = z[:, 24:32]
    m = jnp.maximum(jnp.maximum(x0, x1), jnp.maximum(x2, x3))
    s = (jnp.exp(x0 - m) + jnp.exp(x1 - m) +
         jnp.exp(x2 - m) + jnp.exp(x3 - m))
    lse = m + jnp.log(s)
    out_ref[...] = jnp.concatenate(
        [jnp.maximum(x0 - lse, MIN_LL), jnp.maximum(x1 - lse, MIN_LL),
         jnp.maximum(x2 - lse, MIN_LL), jnp.maximum(x3 - lse, MIN_LL)],
        axis=1)


def _build_table(mu, random_effects):
    mu32 = jnp.tile(mu.T.reshape(1, K * D), (8, 1)).astype(jnp.float32)
    re32 = jnp.repeat(random_effects.astype(jnp.float32), K, axis=1)
    table = pl.pallas_call(
        _table_body,
        grid=(A // _TBLK,),
        in_specs=[
            pl.BlockSpec((8, K * D), lambda i: (0, 0)),
            pl.BlockSpec((_TBLK, K * D), lambda i: (i, 0)),
        ],
        out_specs=pl.BlockSpec((_TBLK, K * D), lambda i: (i, 0)),
        out_shape=jax.ShapeDtypeStruct((A, K * D), jnp.float32),
    )(mu32, re32)
    return table.reshape(A * D, K)


_NC = 2
_NS = 16
_L = 16
_NW = _NC * _NS
_W = N // _NW
_C = 4096
_NCHUNK = _W // _C
_GPC = _C // _L
_JD = _C // 128


def _sc_gather_body(llc, annotators, anno, out, ann_v, anno_v, idx_v, g_v, sem):
    wid = lax.axis_index("s") * _NC + lax.axis_index("c")
    for ci in range(_NCHUNK):
        base = wid * _W + ci * _C
        pltpu.sync_copy(annotators.at[pl.ds(base, _C)], ann_v)
        pltpu.sync_copy(anno.at[pl.ds(base, _C)], anno_v)

        def idx_body(i, _):
            sl = pl.ds(i * _L, _L)
            idx_v[sl] = ann_v[sl] * D + anno_v[sl]
            return 0

        lax.fori_loop(0, _GPC, idx_body, 0)

        cps = []
        for j in range(_JD):
            cps.append(pltpu.async_copy(
                llc.at[idx_v.at[pl.ds(j * 128, 128)]],
                g_v.at[pl.ds(j * 128, 128)],
                sem))
        for cp in cps:
            cp.wait()

        pltpu.sync_copy(g_v, out.at[pl.ds(base, _C)])


@functools.lru_cache(maxsize=1)
def _make_sc_gather():
    mesh = plsc.VectorSubcoreMesh(core_axis_name="c", subcore_axis_name="s")
    return pl.kernel(
        _sc_gather_body,
        mesh=mesh,
        compiler_params=pltpu.CompilerParams(use_tc_tiling_on_sc=False),
        out_type=jax.ShapeDtypeStruct((N, K), jnp.float32),
        scratch_types=[
            pltpu.VMEM((_C,), jnp.int32),
            pltpu.VMEM((_C,), jnp.int32),
            pltpu.VMEM((_C,), jnp.int32),
            pltpu.VMEM((_C, K), jnp.float32),
            pltpu.SemaphoreType.DMA,
        ],
    )


_SBLK = 8192


def _scale_body(g_ref, cf_ref, out_ref):
    row = lax.broadcasted_iota(jnp.int32, (K, K), 0)
    col = lax.broadcasted_iota(jnp.int32, (K, K), 1)
    eye = (row == col).astype(jnp.float32)
    t = lax.dot_general(eye, g_ref[...],
                        dimension_numbers=(((0,), (1,)), ((), ())),
                        preferred_element_type=jnp.float32,
                        precision=lax.Precision.HIGHEST)
    out_ref[...] = t * cf_ref[...]


def _scale(gth, confidence):
    cf = confidence.astype(jnp.float32).reshape(1, N)
    return pl.pallas_call(
        _scale_body,
        grid=(N // _SBLK,),
        in_specs=[
            pl.BlockSpec((_SBLK, K), lambda i: (i, 0)),
            pl.BlockSpec((1, _SBLK), lambda i: (0, i)),
        ],
        out_specs=pl.BlockSpec((K, _SBLK), lambda i: (0, i)),
        out_shape=jax.ShapeDtypeStruct((K, N), jnp.float32),
    )(gth, cf)


def kernel(mu, anno, annotators, confidence, random_effects):
    llc = _build_table(mu, random_effects)
    llt = _make_sc_gather()(llc,
                            annotators.astype(jnp.int32),
                            anno.astype(jnp.int32))
    return _scale(llt, confidence)


# speedup vs baseline: 10.0197x; 1.0688x over previous
import functools
import math

import jax
import jax.numpy as jnp
from jax import lax
from jax.experimental import pallas as pl
from jax.experimental.pallas import tpu as pltpu
from jax.experimental.pallas import tpu_sc as plsc

K = 8
D = 4
N = 262144
A = 16384
MIN_LL = math.log(1e-06)


_TBLK = 1024


def _table_body(mu_ref, re_ref, out_ref):
    z = jnp.exp(mu_ref[0:1, :]) + re_ref[...]
    xd = [jnp.concatenate([z[:, m * 32 + d * 8:m * 32 + d * 8 + 8]
                           for m in range(4)], axis=1)
          for d in range(D)]
    mx = jnp.maximum(jnp.maximum(xd[0], xd[1]), jnp.maximum(xd[2], xd[3]))
    s = sum(jnp.exp(x - mx) for x in xd)
    lse = mx + jnp.log(s)
    ll = [jnp.maximum(x - lse, MIN_LL) for x in xd]
    out_ref[...] = jnp.concatenate(
        [ll[d][:, m * 8:m * 8 + 8] for m in range(4) for d in range(D)],
        axis=1)


def _build_table(mu, random_effects):
    mu128 = jnp.tile(mu.T.reshape(1, K * D), (8, 4)).astype(jnp.float32)
    re128 = jnp.repeat(random_effects.astype(jnp.float32), K,
                       axis=1).reshape(A // 4, 128)
    table = pl.pallas_call(
        _table_body,
        grid=(A // 4 // _TBLK,),
        in_specs=[
            pl.BlockSpec((8, 128), lambda i: (0, 0)),
            pl.BlockSpec((_TBLK, 128), lambda i: (i, 0)),
        ],
        out_specs=pl.BlockSpec((_TBLK, 128), lambda i: (i, 0)),
        out_shape=jax.ShapeDtypeStruct((A // 4, 128), jnp.float32),
    )(mu128, re128)
    return table.reshape(A * D, K)


_NC = 2
_NS = 16
_L = 16
_NW = _NC * _NS
_W = N // _NW
_C = 4096
_R = _C // 16
_NCHUNK = _W // _C
_GPC = _C // _L
_JD = _C // 128


def _sc_gather_body(llc, annotators, anno, out, ann_v, anno_v, comb_v, idx_v,
                    g_v, sem):
    wid = lax.axis_index("s") * _NC + lax.axis_index("c")
    iota = lax.iota(jnp.int32, _L)
    ior = iota * _R
    for ci in range(_NCHUNK):
        base = wid * _W + ci * _C
        pltpu.sync_copy(annotators.at[pl.ds(base, _C)], ann_v)
        pltpu.sync_copy(anno.at[pl.ds(base, _C)], anno_v)

        def comb_body(i, _):
            sl = pl.ds(i * _L, _L)
            comb_v[sl] = ann_v[sl] * D + anno_v[sl]
            return 0

        lax.fori_loop(0, _GPC, comb_body, 0)

        def perm_body(i, _):
            idx_v[pl.ds(i * _L, _L)] = plsc.load_gather(comb_v, [ior + i])
            return 0

        lax.fori_loop(0, _GPC, perm_body, 0)

        cps = []
        for j in range(_JD):
            cps.append(pltpu.async_copy(
                llc.at[idx_v.at[pl.ds(j * 128, 128)]],
                g_v.at[pl.ds(j * 128, 128)],
                sem))
        for cp in cps:
            cp.wait()

        pltpu.sync_copy(g_v, out.at[pl.ds(base, _C)])


@functools.lru_cache(maxsize=1)
def _make_sc_gather():
    mesh = plsc.VectorSubcoreMesh(core_axis_name="c", subcore_axis_name="s")
    return pl.kernel(
        _sc_gather_body,
        mesh=mesh,
        compiler_params=pltpu.CompilerParams(use_tc_tiling_on_sc=False,
                                             needs_layout_passes=False),
        out_type=jax.ShapeDtypeStruct((N, K), jnp.float32),
        scratch_types=[
            pltpu.VMEM((_C,), jnp.int32),
            pltpu.VMEM((_C,), jnp.int32),
            pltpu.VMEM((_C,), jnp.int32),
            pltpu.VMEM((_C,), jnp.int32),
            pltpu.VMEM((_C, K), jnp.float32),
            pltpu.SemaphoreType.DMA,
        ],
    )


def _scale_body(g_ref, cf_ref, out_ref):
    x = g_ref[...]
    row = lax.broadcasted_iota(jnp.int32, (K, K), 0)
    col = lax.broadcasted_iota(jnp.int32, (K, K), 1)
    eye = (row == col).astype(jnp.float32)
    cols = []
    for u in range(16):
        xu = x[:, u * 8:u * 8 + 8]
        tu = lax.dot_general(eye, xu, (((0,), (1,)), ((), ())),
                             preferred_element_type=jnp.float32,
                             precision=lax.Precision.HIGHEST)
        cols.append(tu)
    out_ref[...] = jnp.concatenate(cols, axis=1) * cf_ref[...]


def _scale(gth, confidence):
    g2 = gth.reshape(N // 16, 128)
    cf = confidence.astype(jnp.float32).reshape(1, N)
    return pl.pallas_call(
        _scale_body,
        grid=(N // _C,),
        in_specs=[
            pl.BlockSpec((_R, 128), lambda i: (i, 0)),
            pl.BlockSpec((1, _C), lambda i: (0, i)),
        ],
        out_specs=pl.BlockSpec((K, _C), lambda i: (0, i)),
        out_shape=jax.ShapeDtypeStruct((K, N), jnp.float32),
    )(g2, cf)


def kernel(mu, anno, annotators, confidence, random_effects):
    llc = _build_table(mu, random_effects)
    gth = _make_sc_gather()(llc,
                            annotators.astype(jnp.int32),
                            anno.astype(jnp.int32))
    return _scale(gth, confidence)


# speedup vs baseline: 25.7676x; 2.5717x over previous
import functools
import math

import jax
import jax.numpy as jnp
from jax import lax
from jax.experimental import pallas as pl
from jax.experimental.pallas import tpu as pltpu
from jax.experimental.pallas import tpu_sc as plsc

K = 8
D = 4
N = 262144
A = 16384
MIN_LL = math.log(1e-06)


_TBLK = 1024


def _table_body(mu_ref, re_ref, out_ref):
    q = lax.broadcasted_iota(jnp.int32, (16, 128), 0)
    j = lax.broadcasted_iota(jnp.int32, (16, 128), 1)
    spread = (j // K == q).astype(jnp.float32)
    re_sp = lax.dot_general(re_ref[...], spread, (((1,), (0,)), ((), ())),
                            preferred_element_type=jnp.float32,
                            precision=lax.Precision.HIGHEST)
    z = jnp.exp(mu_ref[0:1, :]) + re_sp

    def back(x, s):
        return pltpu.roll(x, 128 - s, axis=1)

    def fwd(x, s):
        return pltpu.roll(x, s, axis=1)

    lane = lax.broadcasted_iota(jnp.int32, (1, 128), 1)
    g8 = (lane % 32) // K

    def spread_d0(x):
        return jnp.where(g8 == 0, x,
                         jnp.where(g8 == 1, fwd(x, 8),
                                   jnp.where(g8 == 2, fwd(x, 16),
                                             fwd(x, 24))))

    mx0 = jnp.maximum(jnp.maximum(z, back(z, 8)),
                      jnp.maximum(back(z, 16), back(z, 24)))
    mxf = spread_d0(mx0)
    e = jnp.exp(z - mxf)
    s0 = e + back(e, 8) + back(e, 16) + back(e, 24)
    lse = mxf + jnp.log(spread_d0(s0))
    out_ref[...] = jnp.maximum(z - lse, MIN_LL)


def _build_table(mu, random_effects):
    mu128 = jnp.tile(mu.T.reshape(1, K * D), (8, 4)).astype(jnp.float32)
    re16 = random_effects.astype(jnp.float32).reshape(A // 4, 16)
    table = pl.pallas_call(
        _table_body,
        grid=(A // 4 // _TBLK,),
        in_specs=[
            pl.BlockSpec((8, 128), lambda i: (0, 0)),
            pl.BlockSpec((_TBLK, 16), lambda i: (i, 0)),
        ],
        out_specs=pl.BlockSpec((_TBLK, 128), lambda i: (i, 0)),
        out_shape=jax.ShapeDtypeStruct((A // 4, 128), jnp.float32),
    )(mu128, re16)
    return table.reshape(A * D, K)


_NC = 2
_NS = 16
_L = 16
_NW = _NC * _NS
_W = N // _NW
_C = 4096
_NCHUNK = _W // _C
_GPC = _C // _L
_JD = _C // 128


def _sc_gather_body(llc, annotators, anno, out, ann_v, anno_v, comb_v, g_v, sem):
    wid = lax.axis_index("s") * _NC + lax.axis_index("c")
    for ci in range(_NCHUNK):
        base = wid * _W + ci * _C
        pltpu.sync_copy(annotators.at[pl.ds(base, _C)], ann_v)
        pltpu.sync_copy(anno.at[pl.ds(base, _C)], anno_v)

        def comb_body(i, _):
            sl = pl.ds(i * _L, _L)
            comb_v[sl] = ann_v[sl] * D + anno_v[sl]
            return 0

        lax.fori_loop(0, _GPC, comb_body, 0)

        cps = []
        for j in range(_JD):
            cps.append(pltpu.async_copy(
                llc.at[comb_v.at[pl.ds(j * 128, 128)]],
                g_v.at[pl.ds(j * 128, 128)],
                sem))
        for cp in cps:
            cp.wait()

        pltpu.sync_copy(g_v, out.at[pl.ds(base, _C)])


@functools.lru_cache(maxsize=1)
def _make_sc_gather():
    mesh = plsc.VectorSubcoreMesh(core_axis_name="c", subcore_axis_name="s")
    return pl.kernel(
        _sc_gather_body,
        mesh=mesh,
        compiler_params=pltpu.CompilerParams(use_tc_tiling_on_sc=False),
        out_type=jax.ShapeDtypeStruct((N, K), jnp.float32),
        scratch_types=[
            pltpu.VMEM((_C,), jnp.int32),
            pltpu.VMEM((_C,), jnp.int32),
            pltpu.VMEM((_C,), jnp.int32),
            pltpu.VMEM((_C, K), jnp.float32),
            pltpu.SemaphoreType.DMA,
        ],
    )


def _sc_scale_body(g1, conf, out, gv, cf_v, ob_v, sem):
    wid = lax.axis_index("s") * _NC + lax.axis_index("c")
    io = lax.iota(jnp.int32, _L)
    iks = [io * K + kk for kk in range(K)]
    for ci in range(_NCHUNK):
        base = wid * _W + ci * _C
        pltpu.sync_copy(g1.at[pl.ds(base * K, _C * K)], gv)
        pltpu.sync_copy(conf.at[pl.ds(base, _C)], cf_v)

        def body(g, _):
            cv = cf_v[pl.ds(g * _L, _L)]
            for kk in range(K):
                vals = plsc.load_gather(gv, [iks[kk] + g * (K * _L)])
                ob_v[kk, pl.ds(g * _L, _L)] = vals * cv
            return 0

        lax.fori_loop(0, _GPC, body, 0)
        for kk in range(K):
            pltpu.sync_copy(ob_v.at[kk], out.at[kk, pl.ds(base, _C)])


@functools.lru_cache(maxsize=1)
def _make_sc_scale():
    mesh = plsc.VectorSubcoreMesh(core_axis_name="c", subcore_axis_name="s")
    return pl.kernel(
        _sc_scale_body,
        mesh=mesh,
        compiler_params=pltpu.CompilerParams(needs_layout_passes=False),
        out_type=jax.ShapeDtypeStruct((K, N), jnp.float32),
        scratch_types=[
            pltpu.VMEM((_C * K,), jnp.float32),
            pltpu.VMEM((_C,), jnp.float32),
            pltpu.VMEM((K, _C), jnp.float32),
            pltpu.SemaphoreType.DMA,
        ],
    )


def kernel(mu, anno, annotators, confidence, random_effects):
    llc = _build_table(mu, random_effects)
    gth = _make_sc_gather()(llc,
                            annotators.astype(jnp.int32),
                            anno.astype(jnp.int32))
    return _make_sc_scale()(gth.reshape(N * K),
                            confidence.astype(jnp.float32))


# speedup vs baseline: 33.6395x; 1.3055x over previous
import functools
import math

import jax
import jax.numpy as jnp
from jax import lax
from jax.experimental import pallas as pl
from jax.experimental.pallas import tpu as pltpu
from jax.experimental.pallas import tpu_sc as plsc

K = 8
D = 4
N = 262144
A = 16384
MIN_LL = math.log(1e-06)


def _table_body(mu_ref, re_ref, out_ref):
    q = lax.broadcasted_iota(jnp.int32, (D, 32), 0)
    j32 = lax.broadcasted_iota(jnp.int32, (D, 32), 1)
    spread = (j32 // K == q).astype(jnp.float32)
    re_sp = jnp.concatenate(
        [lax.dot_general(re_ref[m * (A // 4):(m + 1) * (A // 4), :], spread,
                         (((1,), (0,)), ((), ())),
                         preferred_element_type=jnp.float32,
                         precision=lax.Precision.HIGHEST)
         for m in range(4)], axis=1)
    z = jnp.exp(mu_ref[0:1, :]) + re_sp

    def back(x, s):
        return pltpu.roll(x, 128 - s, axis=1)

    def fwd(x, s):
        return pltpu.roll(x, s, axis=1)

    lane = lax.broadcasted_iota(jnp.int32, (1, 128), 1)
    g8 = (lane % 32) // K

    def spread_d0(x):
        return jnp.where(g8 == 0, x,
                         jnp.where(g8 == 1, fwd(x, 8),
                                   jnp.where(g8 == 2, fwd(x, 16),
                                             fwd(x, 24))))

    mx0 = jnp.maximum(jnp.maximum(z, back(z, 8)),
                      jnp.maximum(back(z, 16), back(z, 24)))
    mxf = spread_d0(mx0)
    e = jnp.exp(z - mxf)
    s0 = e + back(e, 8) + back(e, 16) + back(e, 24)
    lse = mxf + jnp.log(spread_d0(s0))
    out_ref[...] = jnp.maximum(z - lse, MIN_LL)


def _build_table(mu, random_effects):
    mu128 = jnp.tile(mu.T.reshape(1, K * D), (8, 4)).astype(jnp.float32)
    re = random_effects.astype(jnp.float32)
    table = pl.pallas_call(
        _table_body,
        in_specs=[
            pl.BlockSpec((8, 128), lambda: (0, 0)),
            pl.BlockSpec((A, D), lambda: (0, 0)),
        ],
        out_specs=pl.BlockSpec((A // 4, 128), lambda: (0, 0)),
        out_shape=jax.ShapeDtypeStruct((A // 4, 128), jnp.float32),
    )(mu128, re)
    return table.reshape(A * D, K)


_NC = 2
_NS = 16
_L = 16
_NW = _NC * _NS
_W = N // _NW
_C = 4096
_NCHUNK = _W // _C
_GPC = _C // _L
_JD = _C // 128


def _sc_gather_body(llc, annotators, anno, out,
                    ann0, ann1, anno0, anno1, comb0, comb1, g0, g1,
                    sem_in, sem_g, sem_out):
    wid = lax.axis_index("s") * _NC + lax.axis_index("c")
    anns, annos, combs, gs = [ann0, ann1], [anno0, anno1], [comb0, comb1], [g0, g1]

    def start_in(ci):
        base = wid * _W + ci * _C
        b = ci % 2
        return (pltpu.async_copy(annotators.at[pl.ds(base, _C)], anns[b], sem_in),
                pltpu.async_copy(anno.at[pl.ds(base, _C)], annos[b], sem_in))

    pend_in = start_in(0)
    pend_out = None
    for ci in range(_NCHUNK):
        b = ci % 2
        base = wid * _W + ci * _C
        for cp in pend_in:
            cp.wait()

        ann_v, anno_v, comb_v, g_v = anns[b], annos[b], combs[b], gs[b]

        @plsc.parallel_loop(0, _GPC, unroll=4)
        def comb_body(i):
            sl = pl.ds(i * _L, _L)
            av = ann_v[sl]
            comb_v[sl] = ((av & (A // 4 - 1)) * 16 + (av >> 12) * D
                          + anno_v[sl])

        cps = []
        for j in range(_JD):
            cps.append(pltpu.async_copy(
                llc.at[comb_v.at[pl.ds(j * 128, 128)]],
                g_v.at[pl.ds(j * 128, 128)],
                sem_g))
        if ci + 1 < _NCHUNK:
            pend_in = start_in(ci + 1)
        for cp in cps:
            cp.wait()
        if pend_out is not None:
            pend_out.wait()
        pend_out = pltpu.async_copy(g_v, out.at[pl.ds(base, _C)], sem_out)
    pend_out.wait()


@functools.lru_cache(maxsize=1)
def _make_sc_gather():
    mesh = plsc.VectorSubcoreMesh(core_axis_name="c", subcore_axis_name="s")
    return pl.kernel(
        _sc_gather_body,
        mesh=mesh,
        compiler_params=pltpu.CompilerParams(use_tc_tiling_on_sc=False),
        out_type=jax.ShapeDtypeStruct((N, K), jnp.float32),
        scratch_types=(
            [pltpu.VMEM((_C,), jnp.int32)] * 6 +
            [pltpu.VMEM((_C, K), jnp.float32)] * 2 +
            [pltpu.SemaphoreType.DMA] * 3
        ),
    )


_CB = 2048
_NCHB = _W // _CB
_GPB = _CB // _L


def _sc_scale_body(g1, conf, out, gv0, gv1, cf0, cf1, ob0, ob1,
                   sem_in, sem_out):
    wid = lax.axis_index("s") * _NC + lax.axis_index("c")
    io = lax.iota(jnp.int32, _L)
    iks = [io * K + kk for kk in range(K)]
    gvs, cfs, obs = [gv0, gv1], [cf0, cf1], [ob0, ob1]

    def start_in(ci):
        base = wid * _W + ci * _CB
        b = ci % 2
        return (pltpu.async_copy(g1.at[pl.ds(base * K, _CB * K)], gvs[b], sem_in),
                pltpu.async_copy(conf.at[pl.ds(base, _CB)], cfs[b], sem_in))

    pend_in = start_in(0)
    pend_out = [[], []]
    for ci in range(_NCHB):
        b = ci % 2
        base = wid * _W + ci * _CB
        for cp in pend_in:
            cp.wait()
        if ci + 1 < _NCHB:
            pend_in = start_in(ci + 1)
        for o in pend_out[b]:
            o.wait()
        gv, cf_v, ob_v = gvs[b], cfs[b], obs[b]

        @plsc.parallel_loop(0, _GPB, unroll=4)
        def body(g):
            cv = cf_v[pl.ds(g * _L, _L)]
            for kk in range(K):
                vals = plsc.load_gather(gv, [iks[kk] + g * (K * _L)])
                ob_v[kk, pl.ds(g * _L, _L)] = vals * cv

        pend_out[b] = [pltpu.async_copy(ob_v.at[kk],
                                        out.at[kk, pl.ds(base, _CB)], sem_out)
                       for kk in range(K)]
    for lst in pend_out:
        for o in lst:
            o.wait()


@functools.lru_cache(maxsize=1)
def _make_sc_scale():
    mesh = plsc.VectorSubcoreMesh(core_axis_name="c", subcore_axis_name="s")
    return pl.kernel(
        _sc_scale_body,
        mesh=mesh,
        compiler_params=pltpu.CompilerParams(needs_layout_passes=False),
        out_type=jax.ShapeDtypeStruct((K, N), jnp.float32),
        scratch_types=(
            [pltpu.VMEM((_CB * K,), jnp.float32)] * 2 +
            [pltpu.VMEM((_CB,), jnp.float32)] * 2 +
            [pltpu.VMEM((K, _CB), jnp.float32)] * 2 +
            [pltpu.SemaphoreType.DMA] * 2
        ),
    )


def kernel(mu, anno, annotators, confidence, random_effects):
    llc = _build_table(mu, random_effects)
    gth = _make_sc_gather()(llc,
                            annotators.astype(jnp.int32),
                            anno.astype(jnp.int32))
    return _make_sc_scale()(gth.reshape(N * K),
                            confidence.astype(jnp.float32))
